# baseline (speedup 1.0000x reference)
import functools
import math

import jax
import jax.numpy as jnp
from jax import lax
from jax.experimental import pallas as pl
from jax.experimental.pallas import tpu as pltpu
from jax.experimental.pallas import tpu_sc as plsc

D = 64
SCALE = 8.0

NC = 2
NS = 16
NW = NC * NS

IDX_MINOR = 128
K = 4
CHUNK = K * IDX_MINOR
LANES = 16


def _sc_embed(idx2d, lut):
    n_idx_rows, _ = idx2d.shape
    B = n_idx_rows * IDX_MINOR
    b_per_w = B // NW
    n_chunks = b_per_w // CHUNK
    idx_rows_per_w = b_per_w // IDX_MINOR

    mesh = plsc.VectorSubcoreMesh(core_axis_name="c", subcore_axis_name="s")

    @functools.partial(
        pl.kernel,
        out_type=jax.ShapeDtypeStruct((B, D), jnp.float32),
        mesh=mesh,
        scratch_types=[
            pltpu.VMEM((idx_rows_per_w, IDX_MINOR), jnp.int32),
            pltpu.VMEM((CHUNK, D), jnp.float32),
            pltpu.SemaphoreType.DMA,
        ],
        compiler_params=pltpu.CompilerParams(use_tc_tiling_on_sc=False),
    )
    def k(idx_hbm, table_hbm, out_hbm, idx_v, rows_v, sem):
        wid = lax.axis_index("s") * NC + lax.axis_index("c")
        row_base = pl.multiple_of(wid * b_per_w, 8)
        idx_row_base = pl.multiple_of(wid * idx_rows_per_w, 8)
        pltpu.sync_copy(idx_hbm.at[pl.ds(idx_row_base, idx_rows_per_w)], idx_v)

        def chunk_body(g, carry):
            row0 = pl.multiple_of(row_base + g * CHUNK, 8)
            cps = [
                pltpu.async_copy(
                    table_hbm.at[idx_v.at[g * K + j]],
                    rows_v.at[pl.ds(j * IDX_MINOR, IDX_MINOR)],
                    sem,
                )
                for j in range(K)
            ]
            for cp in cps:
                cp.wait()

            def scale_row(i, c):
                for j in range(D // LANES):
                    sl = pl.ds(j * LANES, LANES)
                    rows_v[i, sl] = rows_v[i, sl] * SCALE
                return c

            lax.fori_loop(0, CHUNK, scale_row, 0)
            pltpu.sync_copy(rows_v, out_hbm.at[pl.ds(row0, CHUNK)])
            return carry

        lax.fori_loop(0, n_chunks, chunk_body, 0)

    return k(idx2d, lut)


def kernel(x, lut):
    S, T = x.shape
    B = S * T
    idx2d = x.reshape(B // IDX_MINOR, IDX_MINOR)
    out = _sc_embed(idx2d, lut)
    return out.reshape(S, T, D)


# speedup vs baseline: 1.1101x; 1.1101x over previous
import functools
import math

import jax
import jax.numpy as jnp
from jax import lax
from jax.experimental import pallas as pl
from jax.experimental.pallas import tpu as pltpu
from jax.experimental.pallas import tpu_sc as plsc

D = 64
SCALE = 8.0

NC = 2
NS = 16
NW = NC * NS

IDX_MINOR = 128
K = 2
CHUNK = K * IDX_MINOR
LANES = 16

NBUF = 5
PREFETCH = 3


def _sc_embed(idx2d, lut):
    n_idx_rows, _ = idx2d.shape
    B = n_idx_rows * IDX_MINOR
    b_per_w = B // NW
    n_chunks = b_per_w // CHUNK
    idx_rows_per_w = b_per_w // IDX_MINOR
    assert n_chunks % NBUF == 0

    mesh = plsc.VectorSubcoreMesh(core_axis_name="c", subcore_axis_name="s")

    @functools.partial(
        pl.kernel,
        out_type=jax.ShapeDtypeStruct((B, D), jnp.float32),
        mesh=mesh,
        scratch_types=[
            pltpu.VMEM((idx_rows_per_w, IDX_MINOR), jnp.int32),
            pltpu.VMEM((NBUF, CHUNK, D), jnp.float32),
            pltpu.SemaphoreType.DMA((NBUF,)),
            pltpu.SemaphoreType.DMA((NBUF,)),
        ],
        compiler_params=pltpu.CompilerParams(use_tc_tiling_on_sc=False),
    )
    def k(idx_hbm, table_hbm, out_hbm, idx_v, rows_v, gsem, osem):
        wid = lax.axis_index("s") * NC + lax.axis_index("c")
        row_base = pl.multiple_of(wid * b_per_w, 8)
        idx_row_base = pl.multiple_of(wid * idx_rows_per_w, 8)
        pltpu.sync_copy(idx_hbm.at[pl.ds(idx_row_base, idx_rows_per_w)], idx_v)

        def fire_gather(f, b):
            for jj in range(K):
                pltpu.async_copy(
                    table_hbm.at[idx_v.at[f * K + jj]],
                    rows_v.at[b, pl.ds(jj * IDX_MINOR, IDX_MINOR)],
                    gsem.at[b],
                )

        def wait_gather(b):
            pltpu.make_async_copy(
                table_hbm.at[pl.ds(0, CHUNK)], rows_v.at[b], gsem.at[b]
            ).wait()

        def wait_out(b):
            pltpu.make_async_copy(
                rows_v.at[b], out_hbm.at[pl.ds(0, CHUNK)], osem.at[b]
            ).wait()

        for p in range(PREFETCH):
            fire_gather(p, p)

        def super_body(s, carry):
            g0 = s * NBUF
            for j in range(NBUF):
                g = g0 + j
                f = g + PREFETCH
                bf = (j + PREFETCH) % NBUF

                @pl.when(jnp.logical_and(f >= NBUF, f < n_chunks))
                def _():
                    wait_out(bf)

                @pl.when(f < n_chunks)
                def _():
                    fire_gather(f, bf)

                wait_gather(j)

                def scale_row(i, c):
                    for q in range(D // LANES):
                        sl = pl.ds(q * LANES, LANES)
                        rows_v[j, i, sl] = rows_v[j, i, sl] * SCALE
                    return c

                lax.fori_loop(0, CHUNK, scale_row, 0)

                row0 = pl.multiple_of(row_base + g * CHUNK, 8)
                pltpu.async_copy(
                    rows_v.at[j], out_hbm.at[pl.ds(row0, CHUNK)], osem.at[j]
                )
            return carry

        lax.fori_loop(0, n_chunks // NBUF, super_body, 0)

        for j in range(NBUF):
            wait_out(j)

    return k(idx2d, lut)


def kernel(x, lut):
    S, T = x.shape
    B = S * T
    idx2d = x.reshape(B // IDX_MINOR, IDX_MINOR)
    out = _sc_embed(idx2d, lut)
    return out.reshape(S, T, D)
